# baseline (device time: 47668 ns/iter reference)
import jax
import jax.numpy as jnp
from jax import lax
from jax.experimental import pallas as pl
from jax.experimental.pallas import tpu as pltpu

CH = 128


def kernel(x, dest):
    m, n = x.shape
    mp = m + 8
    max_chunks = m // CH + 5

    perm = jnp.argsort(dest, stable=True).astype(jnp.int32)
    perm_pad = jnp.concatenate([perm, jnp.zeros((8,), jnp.int32)])
    xs_pad = x[perm_pad].astype(jnp.bfloat16)
    c0 = jnp.sum(dest == 0).astype(jnp.int32).reshape(1)

    def body(c0_ref, xs_hbm, out_ref, recv_ref, xs_vmem, copy_sem,
             send_sems, recv_sems):
        zi = lax.axis_index("z")
        xi = lax.axis_index("x")
        yi = lax.axis_index("y")
        peer = (xi, yi, 1 - zi)
        c0s = c0_ref[0]

        ks = jnp.where(zi == 0, m - c0s, c0s)
        soff = jnp.where(zi == 0, c0s, 0)
        a = pl.multiple_of((soff // 8) * 8, 8)
        r = soff - a
        r_peer = jnp.where(zi == 0, 0, (m - c0s) % 8)
        send_total = ks + r
        recv_total = ks + r_peer

        local_cp = pltpu.make_async_copy(xs_hbm, xs_vmem, copy_sem)
        local_cp.start()

        barrier_sem = pltpu.get_barrier_semaphore()
        pl.semaphore_signal(
            barrier_sem, inc=1, device_id=peer,
            device_id_type=pl.DeviceIdType.MESH,
        )
        pl.semaphore_wait(barrier_sem, 1)

        def chunk_rdma(i, o, size):
            o = pl.multiple_of(o, 8)
            return pltpu.make_async_remote_copy(
                src_ref=xs_hbm.at[pl.ds(pl.multiple_of(a + o, 8), size)],
                dst_ref=recv_ref.at[pl.ds(o, size)],
                send_sem=send_sems.at[i],
                recv_sem=recv_sems.at[i],
                device_id=peer,
                device_id_type=pl.DeviceIdType.MESH,
            )

        def for_each_chunk(total, action):
            nf = total // CH
            rem8 = ((total - nf * CH) + 7) // 8 * 8

            def full_body(i, _):
                action(chunk_rdma(i, i * CH, CH))
                return 0

            lax.fori_loop(0, nf, full_body, 0)
            idx, off = nf, nf * CH
            for b in (128, 64, 32, 16, 8):
                take = (rem8 & b) != 0
                idx_c, off_c = idx, off

                @pl.when(take)
                def _(idx_c=idx_c, off_c=off_c, b=b):
                    action(chunk_rdma(idx_c, off_c, b))

                idx = idx + jnp.where(take, 1, 0)
                off = off + jnp.where(take, b, 0)

        for_each_chunk(send_total, lambda rdma: rdma.start())
        for_each_chunk(recv_total, lambda rdma: rdma.wait_recv())
        local_cp.wait()

        shift = jnp.where(zi == 0, c0s - r_peer, (m - r_peer) % m)
        rolled = pltpu.roll(recv_ref[pl.ds(0, m), :], shift, 0)
        rowid = lax.broadcasted_iota(jnp.int32, (m, n), 0)
        take_xs = (rowid < c0s) == (zi == 0)
        out_ref[:, :] = jnp.where(take_xs, xs_vmem[pl.ds(0, m), :], rolled)

        for_each_chunk(send_total, lambda rdma: rdma.wait_send())

    return pl.pallas_call(
        body,
        out_shape=jax.ShapeDtypeStruct((m, n), jnp.bfloat16),
        in_specs=[
            pl.BlockSpec(memory_space=pltpu.SMEM),
            pl.BlockSpec(memory_space=pl.ANY),
        ],
        out_specs=pl.BlockSpec(memory_space=pltpu.VMEM),
        scratch_shapes=[
            pltpu.VMEM((mp, n), jnp.bfloat16),
            pltpu.VMEM((mp, n), jnp.bfloat16),
            pltpu.SemaphoreType.DMA,
            pltpu.SemaphoreType.DMA((max_chunks,)),
            pltpu.SemaphoreType.DMA((max_chunks,)),
        ],
        compiler_params=pltpu.CompilerParams(collective_id=0),
    )(c0, xs_pad)
